# baseline (speedup 1.0000x reference)
import functools

import jax
import jax.numpy as jnp
from jax import lax
from jax.experimental import pallas as pl
from jax.experimental.pallas import tpu as pltpu
from jax.experimental.pallas import tpu_sc as plsc

B = 16384
H = 50
D = 16
V = 1000000
NC = 2
NS = 16
NW = NC * NS
BAGS_PER_W = B // NW
IDS_PER_STREAM = 800
BAGS_PER_STREAM = IDS_PER_STREAM // H
NSTREAM = BAGS_PER_W // BAGS_PER_STREAM
NBUF = 4

_mesh = plsc.VectorSubcoreMesh(core_axis_name="c", subcore_axis_name="s")


def _tree_sum(vals):
    while len(vals) > 1:
        nxt = [a + b for a, b in zip(vals[::2], vals[1::2])]
        if len(vals) % 2:
            nxt.append(vals[-1])
        vals = nxt
    return vals[0]


@functools.partial(
    pl.kernel,
    out_type=jax.ShapeDtypeStruct((B, D), jnp.float32),
    mesh=_mesh,
    scratch_types=[
        pltpu.VMEM((NSTREAM, IDS_PER_STREAM), jnp.int32),
        pltpu.VMEM((NBUF, IDS_PER_STREAM, D), jnp.float32),
        pltpu.VMEM((BAGS_PER_W, D), jnp.float32),
        [pltpu.SemaphoreType.DMA] * NBUF,
    ],
    compiler_params=pltpu.CompilerParams(use_tc_tiling_on_sc=False),
)
def _emb_bag(ids_hbm, table_hbm, out_hbm, idx_v, rows_v, out_v, sems):
    wid = lax.axis_index("s") * NC + lax.axis_index("c")
    pltpu.sync_copy(ids_hbm.at[pl.ds(wid * NSTREAM, NSTREAM)], idx_v)

    for b in range(NBUF):
        pltpu.async_copy(table_hbm.at[idx_v.at[b]], rows_v.at[b], sems[b])

    def step(s, b):
        pltpu.make_async_copy(
            table_hbm.at[idx_v.at[s]], rows_v.at[b], sems[b]
        ).wait()

        def red(k, carry):
            acc = _tree_sum([rows_v[b, k * H + h] for h in range(H)])
            out_v[BAGS_PER_STREAM * s + k] = acc
            return carry

        lax.fori_loop(0, BAGS_PER_STREAM, red, 0)

    def chunk(i, carry):
        s0 = i * NBUF
        for b in range(NBUF):
            s = s0 + b
            step(s, b)
            pltpu.async_copy(
                table_hbm.at[idx_v.at[s + NBUF]], rows_v.at[b], sems[b]
            )
        return carry

    lax.fori_loop(0, NSTREAM // NBUF - 1, chunk, 0)
    for b in range(NBUF):
        step(NSTREAM - NBUF + b, b)

    pltpu.sync_copy(out_v, out_hbm.at[pl.ds(wid * BAGS_PER_W, BAGS_PER_W)])


def kernel(ids, table):
    ids2 = jnp.minimum(ids, V - 1).reshape(B * H // IDS_PER_STREAM, IDS_PER_STREAM)
    table2 = table * jnp.float32(2.0)
    return _emb_bag(ids2, table2) * jnp.float32(0.5)


# speedup vs baseline: 1.6080x; 1.6080x over previous
import functools

import jax
import jax.numpy as jnp
from jax import lax
from jax.experimental import pallas as pl
from jax.experimental.pallas import tpu as pltpu
from jax.experimental.pallas import tpu_sc as plsc

B = 16384
H = 50
D = 16
V = 1000000
NC = 2
NS = 16
NW = NC * NS

_mesh = plsc.VectorSubcoreMesh(core_axis_name="c", subcore_axis_name="s")

NTC = V // 128
TC_W = 244
TC_EXTRA = NTC - NW * TC_W
DBUF = 2


@functools.partial(
    pl.kernel,
    out_type=jax.ShapeDtypeStruct((V // 8, 128), jnp.float32),
    mesh=_mesh,
    scratch_types=[
        pltpu.VMEM((DBUF, 16, 128), jnp.float32),
        pltpu.VMEM((DBUF, 16, 128), jnp.float32),
        [pltpu.SemaphoreType.DMA] * DBUF,
        [pltpu.SemaphoreType.DMA] * DBUF,
    ],
    compiler_params=pltpu.CompilerParams(
        use_tc_tiling_on_sc=True, needs_layout_passes=False
    ),
)
def _transpose(tabt_hbm, out_hbm, in_v, out_v, isems, osems):
    wid = lax.axis_index("s") * NC + lax.axis_index("c")
    t0 = wid * TC_W

    lanes = lax.iota(jnp.int32, 16)

    def start_in(t, b):
        for a in range(2):
            pltpu.async_copy(
                tabt_hbm.at[pl.ds(8 * a, 8), pl.ds(128 * t, 128)],
                in_v.at[b, pl.ds(8 * a, 8)],
                isems[b],
            )

    def wait_in(t, b):
        for a in range(2):
            pltpu.make_async_copy(
                tabt_hbm.at[pl.ds(8 * a, 8), pl.ds(128 * t, 128)],
                in_v.at[b, pl.ds(8 * a, 8)],
                isems[b],
            ).wait()

    def start_out(t, b):
        pltpu.async_copy(
            out_v.at[b], out_hbm.at[pl.ds(16 * t, 16)], osems[b]
        )

    def wait_out(t, b):
        pltpu.make_async_copy(
            out_v.at[b], out_hbm.at[pl.ds(16 * t, 16)], osems[b]
        ).wait()

    def trans(b, ncol=128):
        for k in range(ncol):
            row = plsc.load_gather(
                in_v.at[b], [lanes, jnp.full((16,), k, jnp.int32)]
            )
            out_v[b, k // 8, pl.ds(16 * (k % 8), 16)] = row

    for b in range(DBUF):
        start_in(t0 + b, b)

    def body(i, carry):
        for b in range(DBUF):
            t = t0 + i * DBUF + b
            wait_in(t, b)

            @pl.when(i > 0)
            def _():
                wait_out(t - DBUF, b)

            trans(b)
            start_out(t, b)
            start_in(t + DBUF, b)
        return carry

    lax.fori_loop(0, TC_W // DBUF - 1, body, 0)

    for b in range(DBUF):
        t = t0 + TC_W - DBUF + b
        wait_in(t, b)
        wait_out(t - DBUF, b)
        trans(b)
        start_out(t, b)
    for b in range(DBUF):
        wait_out(t0 + TC_W - DBUF + b, b)

    @pl.when(wid < TC_EXTRA)
    def _():
        t = NW * TC_W + wid
        start_in(t, 0)
        wait_in(t, 0)
        trans(0)
        start_out(t, 0)
        wait_out(t, 0)


BAGS_PER_W = B // NW
IDS_PER_STREAM = 800
BAGS_PER_STREAM = IDS_PER_STREAM // H
NSTREAM = BAGS_PER_W // BAGS_PER_STREAM
NBUF = 4


def _tree_sum(vals):
    while len(vals) > 1:
        nxt = [a + b for a, b in zip(vals[::2], vals[1::2])]
        if len(vals) % 2:
            nxt.append(vals[-1])
        vals = nxt
    return vals[0]


@functools.partial(
    pl.kernel,
    out_type=jax.ShapeDtypeStruct((B, D), jnp.float32),
    mesh=_mesh,
    scratch_types=[
        pltpu.VMEM((NSTREAM, IDS_PER_STREAM), jnp.int32),
        pltpu.VMEM((NBUF, IDS_PER_STREAM, D), jnp.float32),
        pltpu.VMEM((BAGS_PER_W, D), jnp.float32),
        [pltpu.SemaphoreType.DMA] * NBUF,
    ],
    compiler_params=pltpu.CompilerParams(use_tc_tiling_on_sc=False),
)
def _emb_bag(ids_hbm, table_hbm, out_hbm, idx_v, rows_v, out_v, sems):
    wid = lax.axis_index("s") * NC + lax.axis_index("c")
    pltpu.sync_copy(ids_hbm.at[pl.ds(wid * NSTREAM, NSTREAM)], idx_v)

    for b in range(NBUF):
        pltpu.async_copy(table_hbm.at[idx_v.at[b]], rows_v.at[b], sems[b])

    def step(s, b):
        pltpu.make_async_copy(
            table_hbm.at[idx_v.at[s]], rows_v.at[b], sems[b]
        ).wait()

        def red(k, carry):
            acc = _tree_sum([rows_v[b, k * H + h] for h in range(H)])
            out_v[BAGS_PER_STREAM * s + k] = acc
            return carry

        lax.fori_loop(0, BAGS_PER_STREAM, red, 0)

    def chunk(i, carry):
        s0 = i * NBUF
        for b in range(NBUF):
            s = s0 + b
            step(s, b)
            pltpu.async_copy(
                table_hbm.at[idx_v.at[s + NBUF]], rows_v.at[b], sems[b]
            )
        return carry

    lax.fori_loop(0, NSTREAM // NBUF - 1, chunk, 0)
    for b in range(NBUF):
        step(NSTREAM - NBUF + b, b)

    pltpu.sync_copy(out_v, out_hbm.at[pl.ds(wid * BAGS_PER_W, BAGS_PER_W)])


def kernel(ids, table):
    ids2 = jnp.minimum(ids, V - 1).reshape(B * H // IDS_PER_STREAM, IDS_PER_STREAM)
    table_l = _transpose(table.T)
    tail = table[V - 64 :, :].reshape(8, 128)
    table_l = lax.dynamic_update_slice(table_l, tail, (V // 8 - 8, 0))
    return _emb_bag(ids2, table_l.reshape(V, D))


# speedup vs baseline: 1.7213x; 1.0704x over previous
import functools

import jax
import jax.numpy as jnp
from jax import lax
from jax.experimental import pallas as pl
from jax.experimental.pallas import tpu as pltpu
from jax.experimental.pallas import tpu_sc as plsc

B = 16384
H = 50
D = 16
V = 1000000
NC = 2
NS = 16
NW = NC * NS

_mesh = plsc.VectorSubcoreMesh(core_axis_name="c", subcore_axis_name="s")

NTC = V // 128
TC_W = 244
TC_EXTRA = NTC - NW * TC_W
DBUF = 4


@functools.partial(
    pl.kernel,
    out_type=jax.ShapeDtypeStruct((V // 8, 128), jnp.float32),
    mesh=_mesh,
    scratch_types=[
        pltpu.VMEM((DBUF, 16, 128), jnp.float32),
        pltpu.VMEM((DBUF, 16, 128), jnp.float32),
        [pltpu.SemaphoreType.DMA] * DBUF,
        [pltpu.SemaphoreType.DMA] * DBUF,
    ],
    compiler_params=pltpu.CompilerParams(
        use_tc_tiling_on_sc=True, needs_layout_passes=False
    ),
)
def _transpose(tabt_hbm, out_hbm, in_v, out_v, isems, osems):
    wid = lax.axis_index("s") * NC + lax.axis_index("c")
    t0 = wid * TC_W

    lanes = lax.iota(jnp.int32, 16)

    def start_in(t, b):
        pltpu.async_copy(
            tabt_hbm.at[:, pl.ds(128 * t, 128)], in_v.at[b], isems[b]
        )

    def wait_in(t, b):
        pltpu.make_async_copy(
            tabt_hbm.at[:, pl.ds(128 * t, 128)], in_v.at[b], isems[b]
        ).wait()

    def start_out(t, b):
        pltpu.async_copy(
            out_v.at[b], out_hbm.at[pl.ds(16 * t, 16)], osems[b]
        )

    def wait_out(t, b):
        pltpu.make_async_copy(
            out_v.at[b], out_hbm.at[pl.ds(16 * t, 16)], osems[b]
        ).wait()

    def trans(b, ncol=128):
        for k in range(ncol):
            row = plsc.load_gather(
                in_v.at[b], [lanes, jnp.full((16,), k, jnp.int32)]
            )
            out_v[b, k // 8, pl.ds(16 * (k % 8), 16)] = row

    for b in range(DBUF):
        start_in(t0 + b, b)

    def body(i, carry):
        for b in range(DBUF):
            t = t0 + i * DBUF + b
            wait_in(t, b)

            @pl.when(i > 0)
            def _():
                wait_out(t - DBUF, b)

            trans(b)
            start_out(t, b)
            start_in(t + DBUF, b)
        return carry

    lax.fori_loop(0, TC_W // DBUF - 1, body, 0)

    for b in range(DBUF):
        t = t0 + TC_W - DBUF + b
        wait_in(t, b)
        wait_out(t - DBUF, b)
        trans(b)
        start_out(t, b)
    for b in range(DBUF):
        wait_out(t0 + TC_W - DBUF + b, b)

    @pl.when(wid < TC_EXTRA)
    def _():
        t = NW * TC_W + wid
        start_in(t, 0)
        wait_in(t, 0)
        trans(0)
        start_out(t, 0)
        wait_out(t, 0)


BAGS_PER_W = B // NW
IDS_PER_STREAM = 800
BAGS_PER_STREAM = IDS_PER_STREAM // H
NSTREAM = BAGS_PER_W // BAGS_PER_STREAM
NBUF = 4


def _tree_sum(vals):
    while len(vals) > 1:
        nxt = [a + b for a, b in zip(vals[::2], vals[1::2])]
        if len(vals) % 2:
            nxt.append(vals[-1])
        vals = nxt
    return vals[0]


@functools.partial(
    pl.kernel,
    out_type=jax.ShapeDtypeStruct((B, D), jnp.float32),
    mesh=_mesh,
    scratch_types=[
        pltpu.VMEM((NSTREAM, IDS_PER_STREAM), jnp.int32),
        pltpu.VMEM((NBUF, IDS_PER_STREAM, D), jnp.float32),
        pltpu.VMEM((BAGS_PER_W, D), jnp.float32),
        [pltpu.SemaphoreType.DMA] * NBUF,
    ],
    compiler_params=pltpu.CompilerParams(use_tc_tiling_on_sc=False),
)
def _emb_bag(ids_hbm, table_hbm, out_hbm, idx_v, rows_v, out_v, sems):
    wid = lax.axis_index("s") * NC + lax.axis_index("c")
    pltpu.sync_copy(ids_hbm.at[pl.ds(wid * NSTREAM, NSTREAM)], idx_v)

    for b in range(NBUF):
        pltpu.async_copy(table_hbm.at[idx_v.at[b]], rows_v.at[b], sems[b])

    def step(s, b):
        pltpu.make_async_copy(
            table_hbm.at[idx_v.at[s]], rows_v.at[b], sems[b]
        ).wait()

        def red(k, carry):
            acc = _tree_sum([rows_v[b, k * H + h] for h in range(H)])
            out_v[BAGS_PER_STREAM * s + k] = acc
            return carry

        lax.fori_loop(0, BAGS_PER_STREAM, red, 0)

    def chunk(i, carry):
        s0 = i * NBUF
        for b in range(NBUF):
            s = s0 + b
            step(s, b)
            pltpu.async_copy(
                table_hbm.at[idx_v.at[s + NBUF]], rows_v.at[b], sems[b]
            )
        return carry

    lax.fori_loop(0, NSTREAM // NBUF - 1, chunk, 0)
    for b in range(NBUF):
        step(NSTREAM - NBUF + b, b)

    pltpu.sync_copy(out_v, out_hbm.at[pl.ds(wid * BAGS_PER_W, BAGS_PER_W)])


def kernel(ids, table):
    ids2 = jnp.minimum(ids, V - 1).reshape(B * H // IDS_PER_STREAM, IDS_PER_STREAM)
    table_l = _transpose(table.T)
    tail = table[V - 64 :, :].reshape(8, 128)
    table_l = lax.dynamic_update_slice(table_l, tail, (V // 8 - 8, 0))
    return _emb_bag(ids2, table_l.reshape(V, D))


# speedup vs baseline: 2.2918x; 1.3314x over previous
import functools

import jax
import jax.numpy as jnp
from jax import lax
from jax.experimental import pallas as pl
from jax.experimental.pallas import tpu as pltpu
from jax.experimental.pallas import tpu_sc as plsc

B = 16384
H = 50
D = 16
V = 1000000
NC = 2
NS = 16
NW = NC * NS

_mesh = plsc.VectorSubcoreMesh(core_axis_name="c", subcore_axis_name="s")

NTC = V // 128
TC_W = 244
TC_EXTRA = NTC - NW * TC_W
DBUF = 4


@functools.partial(
    pl.kernel,
    out_type=jax.ShapeDtypeStruct((V // 8, 128), jnp.float32),
    mesh=_mesh,
    scratch_types=[
        pltpu.VMEM((DBUF, 16, 128), jnp.float32),
        pltpu.VMEM((DBUF, 16, 128), jnp.float32),
        [pltpu.SemaphoreType.DMA] * DBUF,
        [pltpu.SemaphoreType.DMA] * DBUF,
    ],
    compiler_params=pltpu.CompilerParams(
        use_tc_tiling_on_sc=True, needs_layout_passes=False
    ),
)
def _transpose(tabt_hbm, out_hbm, in_v, out_v, isems, osems):
    wid = lax.axis_index("s") * NC + lax.axis_index("c")
    t0 = wid * TC_W

    lanes = lax.iota(jnp.int32, 16)
    half = lanes < 8
    rot8 = (lanes + 8) % 16
    d_lo = lanes % 8
    d_hi = d_lo + 8

    def start_in(t, b):
        pltpu.async_copy(
            tabt_hbm.at[:, pl.ds(128 * t, 128)], in_v.at[b], isems[b]
        )

    def wait_in(t, b):
        pltpu.make_async_copy(
            tabt_hbm.at[:, pl.ds(128 * t, 128)], in_v.at[b], isems[b]
        ).wait()

    def start_out(t, b):
        pltpu.async_copy(
            out_v.at[b], out_hbm.at[pl.ds(16 * t, 16)], osems[b]
        )

    def wait_out(t, b):
        pltpu.make_async_copy(
            out_v.at[b], out_hbm.at[pl.ds(16 * t, 16)], osems[b]
        ).wait()

    def trans(b, ncol=128):
        for k in range(0, ncol, 2):
            kk = jnp.where(half, k, k + 1)
            ga = plsc.load_gather(in_v.at[b], [d_lo, kk])
            gb = plsc.load_gather(in_v.at[b], [d_hi, kk])
            ra = ga[rot8]
            rb = gb[rot8]
            row0 = jnp.where(half, ga, rb)
            row1 = jnp.where(half, ra, gb)
            out_v[b, k // 8, pl.ds(16 * (k % 8), 16)] = row0
            out_v[b, (k + 1) // 8, pl.ds(16 * ((k + 1) % 8), 16)] = row1

    for b in range(DBUF):
        start_in(t0 + b, b)

    def body(i, carry):
        for b in range(DBUF):
            t = t0 + i * DBUF + b
            wait_in(t, b)

            @pl.when(i > 0)
            def _():
                wait_out(t - DBUF, b)

            trans(b)
            start_out(t, b)
            start_in(t + DBUF, b)
        return carry

    lax.fori_loop(0, TC_W // DBUF - 1, body, 0)

    for b in range(DBUF):
        t = t0 + TC_W - DBUF + b
        wait_in(t, b)
        wait_out(t - DBUF, b)
        trans(b)
        start_out(t, b)
    for b in range(DBUF):
        wait_out(t0 + TC_W - DBUF + b, b)

    @pl.when(wid < TC_EXTRA)
    def _():
        t = NW * TC_W + wid
        start_in(t, 0)
        wait_in(t, 0)
        trans(0)
        start_out(t, 0)
        wait_out(t, 0)


BAGS_PER_W = B // NW
IDS_PER_STREAM = 800
BAGS_PER_STREAM = IDS_PER_STREAM // H
NSTREAM = BAGS_PER_W // BAGS_PER_STREAM
NBUF = 4


def _tree_sum(vals):
    while len(vals) > 1:
        nxt = [a + b for a, b in zip(vals[::2], vals[1::2])]
        if len(vals) % 2:
            nxt.append(vals[-1])
        vals = nxt
    return vals[0]


@functools.partial(
    pl.kernel,
    out_type=jax.ShapeDtypeStruct((B, D), jnp.float32),
    mesh=_mesh,
    scratch_types=[
        pltpu.VMEM((NSTREAM, IDS_PER_STREAM), jnp.int32),
        pltpu.VMEM((NBUF, IDS_PER_STREAM, D), jnp.float32),
        pltpu.VMEM((BAGS_PER_W, D), jnp.float32),
        [pltpu.SemaphoreType.DMA] * NBUF,
    ],
    compiler_params=pltpu.CompilerParams(use_tc_tiling_on_sc=False),
)
def _emb_bag(ids_hbm, table_hbm, out_hbm, idx_v, rows_v, out_v, sems):
    wid = lax.axis_index("s") * NC + lax.axis_index("c")
    pltpu.sync_copy(ids_hbm.at[pl.ds(wid * NSTREAM, NSTREAM)], idx_v)

    for b in range(NBUF):
        pltpu.async_copy(table_hbm.at[idx_v.at[b]], rows_v.at[b], sems[b])

    def step(s, b):
        pltpu.make_async_copy(
            table_hbm.at[idx_v.at[s]], rows_v.at[b], sems[b]
        ).wait()

        def red(k, carry):
            acc = _tree_sum([rows_v[b, k * H + h] for h in range(H)])
            out_v[BAGS_PER_STREAM * s + k] = acc
            return carry

        lax.fori_loop(0, BAGS_PER_STREAM, red, 0)

    def chunk(i, carry):
        s0 = i * NBUF
        for b in range(NBUF):
            s = s0 + b
            step(s, b)
            pltpu.async_copy(
                table_hbm.at[idx_v.at[s + NBUF]], rows_v.at[b], sems[b]
            )
        return carry

    lax.fori_loop(0, NSTREAM // NBUF - 1, chunk, 0)
    for b in range(NBUF):
        step(NSTREAM - NBUF + b, b)

    pltpu.sync_copy(out_v, out_hbm.at[pl.ds(wid * BAGS_PER_W, BAGS_PER_W)])


def kernel(ids, table):
    ids2 = jnp.minimum(ids, V - 1).reshape(B * H // IDS_PER_STREAM, IDS_PER_STREAM)
    table_l = _transpose(table.T)
    tail = table[V - 64 :, :].reshape(8, 128)
    table_l = lax.dynamic_update_slice(table_l, tail, (V // 8 - 8, 0))
    return _emb_bag(ids2, table_l.reshape(V, D))
